# baseline (speedup 1.0000x reference)
import jax
import jax.numpy as jnp
from jax import lax
from jax.experimental import pallas as pl
from jax.experimental.pallas import tpu as pltpu
from jax.experimental.pallas import tpu_sc as plsc

N, E, D, H, OUT = 10000, 320000, 128, 128, 128
NUM_BLOCKS = 3
NC, NS = 2, 16
NW = NC * NS
COLS = H // NC
CHUNK = 128
HALF = 40
HPW = 4
CPT = HALF * HPW
TOT_CHUNKS = NS * CPT
E_PAD = TOT_CHUNKS * CHUNK
ZROWS = 632
ACC_R = NS * ZROWS
EPS = 1e-5


def _agg_body(h2_hbm, src_hbm, dst_hbm, out_hbm,
              hs, acc, src_v, dst_v, rows0, rows1, rows2, rows3,
              sem0, sem1, sem2, sem3, sem4, sem5, sem6, sem7):
    c = lax.axis_index("c")
    s = lax.axis_index("s")

    pltpu.sync_copy(h2_hbm.at[c, pl.ds(s * ZROWS, ZROWS)],
                    hs.at[pl.ds(s * ZROWS, ZROWS)])

    zero16 = jnp.zeros((16,), jnp.float32)

    def zrow(r, carry):
        for k in range(COLS // 16):
            rows0[r, pl.ds(k * 16, 16)] = zero16
        return carry

    lax.fori_loop(0, CHUNK, zrow, 0)
    for j in range(4):
        pltpu.sync_copy(rows0, acc.at[pl.ds(s * ZROWS + j * CHUNK, CHUNK)])
    pltpu.sync_copy(rows0.at[pl.ds(0, ZROWS - 4 * CHUNK)],
                    acc.at[pl.ds(s * ZROWS + 4 * CHUNK, ZROWS - 4 * CHUNK)])
    plsc.subcore_barrier()

    rows = (rows0, rows1, rows2, rows3)
    semg = (sem0, sem1, sem2, sem3)
    sems = (sem4, sem5, sem6, sem7)
    for half in range(HPW):
        base = s * CPT + half * HALF
        pltpu.sync_copy(src_hbm.at[pl.ds(base, HALF)], src_v)
        pltpu.sync_copy(dst_hbm.at[pl.ds(base, HALF)], dst_v)
        for b in range(4):
            pltpu.async_copy(hs.at[src_v.at[b]], rows[b], semg[b])

        def step(i, carry2):
            j0 = 4 * i
            for b in range(4):
                pltpu.make_async_copy(hs.at[src_v.at[j0 + b]], rows[b],
                                      semg[b]).wait()
                pltpu.async_copy(rows[b], acc.at[dst_v.at[j0 + b]], sems[b],
                                 add=True)
            for b in range(4):
                pltpu.make_async_copy(rows[b], acc.at[dst_v.at[j0 + b]],
                                      sems[b]).wait()
                pltpu.async_copy(hs.at[src_v.at[j0 + 4 + b]], rows[b], semg[b])
            return carry2

        lax.fori_loop(0, HALF // 4 - 1, step, 0)
        j0 = HALF - 4
        for b in range(4):
            pltpu.make_async_copy(hs.at[src_v.at[j0 + b]], rows[b],
                                  semg[b]).wait()
            pltpu.async_copy(rows[b], acc.at[dst_v.at[j0 + b]], sems[b],
                             add=True)
        for b in range(4):
            pltpu.make_async_copy(rows[b], acc.at[dst_v.at[j0 + b]],
                                  sems[b]).wait()
    plsc.subcore_barrier()
    pltpu.sync_copy(acc.at[pl.ds(s * ZROWS, ZROWS)],
                    out_hbm.at[c, pl.ds(s * ZROWS, ZROWS)])


_SC_AGG_CACHE = []


def _sc_agg_kernel():
    if not _SC_AGG_CACHE:
        _SC_AGG_CACHE.append(pl.kernel(
            _agg_body,
            out_type=jax.ShapeDtypeStruct((NC, ACC_R, COLS), jnp.float32),
            mesh=plsc.VectorSubcoreMesh(core_axis_name="c",
                                        subcore_axis_name="s",
                                        num_cores=NC, num_subcores=NS),
            compiler_params=pltpu.CompilerParams(use_tc_tiling_on_sc=False),
            scratch_types=[
                pltpu.VMEM_SHARED((ACC_R, COLS), jnp.float32),
                pltpu.VMEM_SHARED((ACC_R, COLS), jnp.float32),
                pltpu.VMEM((HALF, CHUNK), jnp.int32),
                pltpu.VMEM((HALF, CHUNK), jnp.int32),
                pltpu.VMEM((CHUNK, COLS), jnp.float32),
                pltpu.VMEM((CHUNK, COLS), jnp.float32),
                pltpu.VMEM((CHUNK, COLS), jnp.float32),
                pltpu.VMEM((CHUNK, COLS), jnp.float32),
                pltpu.SemaphoreType.DMA,
                pltpu.SemaphoreType.DMA,
                pltpu.SemaphoreType.DMA,
                pltpu.SemaphoreType.DMA,
                pltpu.SemaphoreType.DMA,
                pltpu.SemaphoreType.DMA,
                pltpu.SemaphoreType.DMA,
                pltpu.SemaphoreType.DMA,
            ],
        ))
    return _SC_AGG_CACHE[0]


def _split_halves(h2_ref, h):
    h2_ref[0, :N, :] = h[:, :COLS]
    h2_ref[1, :N, :] = h[:, COLS:]


def _tc_pre_body(x_ref, g_ref, b_ref, w_ref, wb_ref, out_ref, h2_ref):
    x = x_ref[...]
    mu = jnp.mean(x, axis=0, keepdims=True)
    xc = x - mu
    var = jnp.mean(xc * xc, axis=0, keepdims=True)
    xn = xc * lax.rsqrt(var + EPS) * g_ref[...] + b_ref[...]
    h = jnp.dot(xn, w_ref[...], preferred_element_type=jnp.float32) + wb_ref[...]
    h = jnp.maximum(h, 0.0)
    out_ref[...] = h
    _split_halves(h2_ref, h)


def _tc_block_body(h_ref, p_ref, w1_ref, b1_ref, w2_ref, b2_ref, g_ref, be_ref,
                   out_ref, h2_ref):
    h = h_ref[...]
    agg = jnp.concatenate([p_ref[0, :N, :], p_ref[1, :N, :]], axis=1)
    z = h + agg
    z = jnp.maximum(jnp.dot(z, w1_ref[...], preferred_element_type=jnp.float32)
                    + b1_ref[...], 0.0)
    z = jnp.dot(z, w2_ref[...], preferred_element_type=jnp.float32) + b2_ref[...]
    mu = jnp.mean(z, axis=0, keepdims=True)
    zc = z - mu
    var = jnp.mean(zc * zc, axis=0, keepdims=True)
    z = zc * lax.rsqrt(var + EPS) * g_ref[...] + be_ref[...]
    hn = jnp.maximum(z, 0.0) + h
    out_ref[...] = hn
    _split_halves(h2_ref, hn)


def _tc_post_body(h_ref, w_ref, b_ref, emb_ref, log_ref):
    pooled = jnp.mean(h_ref[...], axis=0, keepdims=True)
    pooled8 = jnp.broadcast_to(pooled, (8, H))
    emb_ref[...] = pooled8
    log_ref[...] = (jnp.dot(pooled8, w_ref[...],
                            preferred_element_type=jnp.float32) + b_ref[...])


_H_SHAPES = (jax.ShapeDtypeStruct((N, H), jnp.float32),
             jax.ShapeDtypeStruct((NC, ACC_R, COLS), jnp.float32))


def kernel(x, edge_index, fn_gamma, fn_beta, proj_w, proj_b, blk_w1, blk_b1,
           blk_w2, blk_b2, blk_gamma, blk_beta, pred_w, pred_b):
    f32 = jnp.float32
    src = edge_index[0]
    dst = edge_index[1]
    pad = E_PAD - E
    src_p = jnp.concatenate([src, jnp.zeros((pad,), jnp.int32)])
    dst_p = jnp.concatenate([dst, jnp.full((pad,), N, jnp.int32)])
    src_p = src_p.reshape(TOT_CHUNKS, CHUNK)
    dst_p = dst_p.reshape(TOT_CHUNKS, CHUNK)

    h, h2 = pl.pallas_call(
        _tc_pre_body,
        out_shape=_H_SHAPES,
    )(x, fn_gamma.reshape(1, D), fn_beta.reshape(1, D), proj_w,
      proj_b.reshape(1, H))

    for i in range(NUM_BLOCKS):
        parts = _sc_agg_kernel()(h2, src_p, dst_p)
        h, h2 = pl.pallas_call(
            _tc_block_body,
            out_shape=_H_SHAPES,
        )(h, parts, blk_w1[i], blk_b1[i].reshape(1, H), blk_w2[i],
          blk_b2[i].reshape(1, H), blk_gamma[i].reshape(1, H),
          blk_beta[i].reshape(1, H))

    emb8, log8 = pl.pallas_call(
        _tc_post_body,
        out_shape=(jax.ShapeDtypeStruct((8, H), f32),
                   jax.ShapeDtypeStruct((8, OUT), f32)),
    )(h, pred_w, pred_b.reshape(1, OUT))
    return emb8[:1], log8[:1]


# speedup vs baseline: 1.1512x; 1.1512x over previous
import jax
import jax.numpy as jnp
from jax import lax
from jax.experimental import pallas as pl
from jax.experimental.pallas import tpu as pltpu
from jax.experimental.pallas import tpu_sc as plsc

N, E, D, H, OUT = 10000, 320000, 128, 128, 128
NUM_BLOCKS = 3
NC, NS = 2, 16
NW = NC * NS
COLS = H // NC
CHUNK = 128
HALF = 80
HPW = 2
CPT = HALF * HPW
TOT_CHUNKS = NS * CPT
E_PAD = TOT_CHUNKS * CHUNK
ZROWS = 632
ACC_R = NS * ZROWS
EPS = 1e-5


def _agg_body(h2_hbm, src_hbm, dst_hbm, out_hbm,
              hs, acc, src_v, dst_v, rows0, rows1, sem0, sem1):
    c = lax.axis_index("c")
    s = lax.axis_index("s")

    pltpu.sync_copy(h2_hbm.at[c, pl.ds(s * ZROWS, ZROWS)],
                    hs.at[pl.ds(s * ZROWS, ZROWS)])

    zero16 = jnp.zeros((16,), jnp.float32)

    def zrow(r, carry):
        for k in range(COLS // 16):
            rows0[r, pl.ds(k * 16, 16)] = zero16
        return carry

    lax.fori_loop(0, CHUNK, zrow, 0)
    for j in range(4):
        pltpu.sync_copy(rows0, acc.at[pl.ds(s * ZROWS + j * CHUNK, CHUNK)])
    pltpu.sync_copy(rows0.at[pl.ds(0, ZROWS - 4 * CHUNK)],
                    acc.at[pl.ds(s * ZROWS + 4 * CHUNK, ZROWS - 4 * CHUNK)])
    plsc.subcore_barrier()

    hsrc = hs
    for half in range(HPW):
        base = s * CPT + half * HALF
        pltpu.sync_copy(src_hbm.at[pl.ds(base, HALF)], src_v)
        pltpu.sync_copy(dst_hbm.at[pl.ds(base, HALF)], dst_v)
        pltpu.async_copy(hsrc.at[src_v.at[0]], rows0, sem0)
        pltpu.async_copy(hsrc.at[src_v.at[1]], rows1, sem1)

        def step(i, carry2):
            j0 = 2 * i
            pltpu.make_async_copy(hsrc.at[src_v.at[j0]], rows0, sem0).wait()
            pltpu.sync_copy(rows0, acc.at[dst_v.at[j0]], add=True)
            pltpu.async_copy(hsrc.at[src_v.at[j0 + 2]], rows0, sem0)
            pltpu.make_async_copy(hsrc.at[src_v.at[j0 + 1]], rows1, sem1).wait()
            pltpu.sync_copy(rows1, acc.at[dst_v.at[j0 + 1]], add=True)
            pltpu.async_copy(hsrc.at[src_v.at[j0 + 3]], rows1, sem1)
            return carry2

        lax.fori_loop(0, HALF // 2 - 1, step, 0)
        pltpu.make_async_copy(hsrc.at[src_v.at[HALF - 2]], rows0, sem0).wait()
        pltpu.sync_copy(rows0, acc.at[dst_v.at[HALF - 2]], add=True)
        pltpu.make_async_copy(hsrc.at[src_v.at[HALF - 1]], rows1, sem1).wait()
        pltpu.sync_copy(rows1, acc.at[dst_v.at[HALF - 1]], add=True)
    plsc.subcore_barrier()
    pltpu.sync_copy(acc.at[pl.ds(s * ZROWS, ZROWS)],
                    out_hbm.at[c, pl.ds(s * ZROWS, ZROWS)])


_SC_AGG_CACHE = []


def _sc_agg_kernel():
    if not _SC_AGG_CACHE:
        _SC_AGG_CACHE.append(pl.kernel(
            _agg_body,
            out_type=jax.ShapeDtypeStruct((NC, ACC_R, COLS), jnp.float32),
            mesh=plsc.VectorSubcoreMesh(core_axis_name="c",
                                        subcore_axis_name="s",
                                        num_cores=NC, num_subcores=NS),
            compiler_params=pltpu.CompilerParams(use_tc_tiling_on_sc=False),
            scratch_types=[
                pltpu.VMEM_SHARED((ACC_R, COLS), jnp.float32),
                pltpu.VMEM_SHARED((ACC_R, COLS), jnp.float32),
                pltpu.VMEM((HALF, CHUNK), jnp.int32),
                pltpu.VMEM((HALF, CHUNK), jnp.int32),
                pltpu.VMEM((CHUNK, COLS), jnp.float32),
                pltpu.VMEM((CHUNK, COLS), jnp.float32),
                pltpu.SemaphoreType.DMA,
                pltpu.SemaphoreType.DMA,
            ],
        ))
    return _SC_AGG_CACHE[0]


def _split_halves(h2_ref, h):
    h2_ref[0, :N, :] = h[:, :COLS]
    h2_ref[1, :N, :] = h[:, COLS:]


def _tc_pre_body(x_ref, g_ref, b_ref, w_ref, wb_ref, out_ref, h2_ref):
    x = x_ref[...]
    mu = jnp.mean(x, axis=0, keepdims=True)
    xc = x - mu
    var = jnp.mean(xc * xc, axis=0, keepdims=True)
    xn = xc * lax.rsqrt(var + EPS) * g_ref[...] + b_ref[...]
    h = jnp.dot(xn, w_ref[...], preferred_element_type=jnp.float32) + wb_ref[...]
    h = jnp.maximum(h, 0.0)
    out_ref[...] = h
    _split_halves(h2_ref, h)


def _tc_block_body(h_ref, p_ref, w1_ref, b1_ref, w2_ref, b2_ref, g_ref, be_ref,
                   out_ref, h2_ref):
    h = h_ref[...]
    agg = jnp.concatenate([p_ref[0, :N, :], p_ref[1, :N, :]], axis=1)
    z = h + agg
    z = jnp.maximum(jnp.dot(z, w1_ref[...], preferred_element_type=jnp.float32)
                    + b1_ref[...], 0.0)
    z = jnp.dot(z, w2_ref[...], preferred_element_type=jnp.float32) + b2_ref[...]
    mu = jnp.mean(z, axis=0, keepdims=True)
    zc = z - mu
    var = jnp.mean(zc * zc, axis=0, keepdims=True)
    z = zc * lax.rsqrt(var + EPS) * g_ref[...] + be_ref[...]
    hn = jnp.maximum(z, 0.0) + h
    out_ref[...] = hn
    _split_halves(h2_ref, hn)


def _tc_post_body(h_ref, w_ref, b_ref, emb_ref, log_ref):
    pooled = jnp.mean(h_ref[...], axis=0, keepdims=True)
    pooled8 = jnp.broadcast_to(pooled, (8, H))
    emb_ref[...] = pooled8
    log_ref[...] = (jnp.dot(pooled8, w_ref[...],
                            preferred_element_type=jnp.float32) + b_ref[...])


_H_SHAPES = (jax.ShapeDtypeStruct((N, H), jnp.float32),
             jax.ShapeDtypeStruct((NC, ACC_R, COLS), jnp.float32))


def kernel(x, edge_index, fn_gamma, fn_beta, proj_w, proj_b, blk_w1, blk_b1,
           blk_w2, blk_b2, blk_gamma, blk_beta, pred_w, pred_b):
    f32 = jnp.float32
    src = edge_index[0]
    dst = edge_index[1]
    pad = E_PAD - E
    src_p = jnp.concatenate([src, jnp.zeros((pad,), jnp.int32)])
    dst_p = jnp.concatenate([dst, jnp.full((pad,), N, jnp.int32)])
    src_p = src_p.reshape(TOT_CHUNKS, CHUNK)
    dst_p = dst_p.reshape(TOT_CHUNKS, CHUNK)

    h, h2 = pl.pallas_call(
        _tc_pre_body,
        out_shape=_H_SHAPES,
    )(x, fn_gamma.reshape(1, D), fn_beta.reshape(1, D), proj_w,
      proj_b.reshape(1, H))

    for i in range(NUM_BLOCKS):
        parts = _sc_agg_kernel()(h2, src_p, dst_p)
        h, h2 = pl.pallas_call(
            _tc_block_body,
            out_shape=_H_SHAPES,
        )(h, parts, blk_w1[i], blk_b1[i].reshape(1, H), blk_w2[i],
          blk_b2[i].reshape(1, H), blk_gamma[i].reshape(1, H),
          blk_beta[i].reshape(1, H))

    emb8, log8 = pl.pallas_call(
        _tc_post_body,
        out_shape=(jax.ShapeDtypeStruct((8, H), f32),
                   jax.ShapeDtypeStruct((8, OUT), f32)),
    )(h, pred_w, pred_b.reshape(1, OUT))
    return emb8[:1], log8[:1]
